# Initial kernel scaffold; baseline (speedup 1.0000x reference)
import jax
import jax.numpy as jnp
from jax.experimental import pallas as pl


def kernel(x_variables, x_factors, senders, receivers, edge_attr, W_msg, b_msg, W_comb, b_comb):
    raise NotImplementedError("write your pallas kernel here")


# speedup vs baseline: 2.9850x; 2.9850x over previous
import dataclasses
import functools

import jax
import jax.numpy as jnp
from jax import lax
from jax.experimental import pallas as pl
from jax.experimental.pallas import tpu as pltpu
from jax.experimental.pallas import tpu_sc as plsc

F = 10000
V = 10000
E = 320000
D = 128
NC = 2
NS = 16
L = 16
NW = NC * NS
EPT = E // NW
CHUNK = 80
NCHUNK = EPT // CHUNK
ROWS_PT = 624
ZROWS = 208
TAIL = F - NS * ROWS_PT


def _sc_edge_aggregate(A, B, senders, receivers, edge_attr, w_row):
    mesh = plsc.VectorSubcoreMesh(core_axis_name="c", subcore_axis_name="s")
    cp = pltpu.CompilerParams()
    if "needs_layout_passes" in pltpu.CompilerParams.__dataclass_fields__:
        cp = dataclasses.replace(cp, needs_layout_passes=False)

    @functools.partial(
        pl.kernel,
        out_type=jax.ShapeDtypeStruct((NC, F, D), jnp.float32),
        mesh=mesh,
        compiler_params=cp,
        scratch_types=[
            pltpu.VMEM((CHUNK,), jnp.int32),
            pltpu.VMEM((CHUNK,), jnp.int32),
            pltpu.VMEM((CHUNK,), jnp.float32),
            pltpu.VMEM((CHUNK, D), jnp.float32),
            pltpu.VMEM((CHUNK, D), jnp.float32),
            pltpu.VMEM((D,), jnp.float32),
            pltpu.VMEM((ZROWS, D), jnp.float32),
            pltpu.VMEM_SHARED((F, D), jnp.float32),
            pltpu.SemaphoreType.DMA,
            pltpu.SemaphoreType.DMA,
        ],
    )
    def k(a_hbm, b_hbm, s_hbm, r_hbm, e_hbm, w_hbm, out_hbm,
          sidx, ridx, attr, abuf, bbuf, wv, zbuf, acc, sem_a, sem_b):
        cid = lax.axis_index("c")
        sid = lax.axis_index("s")
        wid = cid * NS + sid

        pltpu.sync_copy(w_hbm, wv)

        zeros = jnp.zeros((L,), jnp.float32)

        @pl.loop(0, ZROWS * (D // L))
        def _(i):
            zbuf[i // (D // L), pl.ds((i % (D // L)) * L, L)] = zeros

        for j in range(ROWS_PT // ZROWS):
            pltpu.sync_copy(zbuf, acc.at[pl.ds(sid * ROWS_PT + j * ZROWS, ZROWS)])

        @pl.when(sid == 0)
        def _():
            pltpu.sync_copy(zbuf.at[pl.ds(0, TAIL)],
                            acc.at[pl.ds(NS * ROWS_PT, TAIL)])

        plsc.subcore_barrier()

        @pl.loop(0, NCHUNK)
        def _(ci):
            base = wid * EPT + ci * CHUNK
            pltpu.sync_copy(s_hbm.at[pl.ds(base, CHUNK)], sidx)
            pltpu.sync_copy(r_hbm.at[pl.ds(base, CHUNK)], ridx)
            pltpu.sync_copy(e_hbm.at[pl.ds(base, CHUNK)], attr)
            cp_a = pltpu.async_copy(a_hbm.at[ridx], abuf, sem_a)
            cp_b = pltpu.async_copy(b_hbm.at[sidx], bbuf, sem_b)
            cp_a.wait()
            cp_b.wait()

            @pl.loop(0, CHUNK)
            def _(e):
                av = plsc.load_gather(attr, [jnp.full((L,), e, jnp.int32)])
                for g in range(D // L):
                    sl = pl.ds(g * L, L)
                    m = abuf[e, sl] + bbuf[e, sl] + av * wv[sl]
                    abuf[e, sl] = jnp.maximum(m, 0.0)

            pltpu.sync_copy(abuf, acc.at[ridx], add=True)

        plsc.subcore_barrier()
        for j in range(ROWS_PT // ZROWS):
            r0 = sid * ROWS_PT + j * ZROWS
            pltpu.sync_copy(acc.at[pl.ds(r0, ZROWS)],
                            out_hbm.at[cid, pl.ds(r0, ZROWS)])

        @pl.when(sid == 0)
        def _():
            pltpu.sync_copy(acc.at[pl.ds(NS * ROWS_PT, TAIL)],
                            out_hbm.at[cid, pl.ds(NS * ROWS_PT, TAIL)])

    return k(A, B, senders, receivers, edge_attr, w_row)


_BLK = 2000


def _tc_precompute(x_factors, x_variables, W1, W2, b_msg):
    def body(xf_ref, xv_ref, w1_ref, w2_ref, b_ref, a_ref, b_out_ref):
        a_ref[...] = jnp.dot(xf_ref[...], w1_ref[...],
                             preferred_element_type=jnp.float32,
                             precision=lax.Precision.HIGHEST) + b_ref[...]
        b_out_ref[...] = jnp.dot(xv_ref[...], w2_ref[...],
                                 preferred_element_type=jnp.float32,
                                 precision=lax.Precision.HIGHEST)

    return pl.pallas_call(
        body,
        grid=(F // _BLK,),
        in_specs=[
            pl.BlockSpec((_BLK, D), lambda i: (i, 0)),
            pl.BlockSpec((_BLK, D), lambda i: (i, 0)),
            pl.BlockSpec((D, D), lambda i: (0, 0)),
            pl.BlockSpec((D, D), lambda i: (0, 0)),
            pl.BlockSpec((1, D), lambda i: (0, 0)),
        ],
        out_specs=[
            pl.BlockSpec((_BLK, D), lambda i: (i, 0)),
            pl.BlockSpec((_BLK, D), lambda i: (i, 0)),
        ],
        out_shape=[
            jax.ShapeDtypeStruct((F, D), jnp.float32),
            jax.ShapeDtypeStruct((V, D), jnp.float32),
        ],
    )(x_factors, x_variables, W1, W2, b_msg.reshape(1, D))


def _tc_combine(x_factors, partials, Wc1, Wc2, b_comb):
    def body(xf_ref, p_ref, w1_ref, w2_ref, b_ref, o_ref):
        aggr = p_ref[0] + p_ref[1]
        acc = jnp.dot(xf_ref[...], w1_ref[...],
                      preferred_element_type=jnp.float32,
                      precision=lax.Precision.HIGHEST)
        acc += jnp.dot(aggr, w2_ref[...],
                       preferred_element_type=jnp.float32,
                       precision=lax.Precision.HIGHEST)
        o_ref[...] = jnp.maximum(acc + b_ref[...], 0.0)

    return pl.pallas_call(
        body,
        grid=(F // _BLK,),
        in_specs=[
            pl.BlockSpec((_BLK, D), lambda i: (i, 0)),
            pl.BlockSpec((NC, _BLK, D), lambda i: (0, i, 0)),
            pl.BlockSpec((D, D), lambda i: (0, 0)),
            pl.BlockSpec((D, D), lambda i: (0, 0)),
            pl.BlockSpec((1, D), lambda i: (0, 0)),
        ],
        out_specs=pl.BlockSpec((_BLK, D), lambda i: (i, 0)),
        out_shape=jax.ShapeDtypeStruct((F, D), jnp.float32),
    )(x_factors, partials, Wc1, Wc2, b_comb.reshape(1, D))


def kernel(x_variables, x_factors, senders, receivers, edge_attr,
           W_msg, b_msg, W_comb, b_comb):
    W1 = W_msg[:D]
    W2 = W_msg[D:2 * D]
    w_row = W_msg[2 * D]
    A, B = _tc_precompute(x_factors, x_variables, W1, W2, b_msg)
    partials = _sc_edge_aggregate(
        A, B,
        senders.astype(jnp.int32), receivers.astype(jnp.int32),
        edge_attr, w_row)
    return _tc_combine(x_factors, partials, W_comb[:D], W_comb[D:], b_comb)


# speedup vs baseline: 7.2282x; 2.4215x over previous
import dataclasses
import functools

import jax
import jax.numpy as jnp
from jax import lax
from jax.experimental import pallas as pl
from jax.experimental.pallas import tpu as pltpu
from jax.experimental.pallas import tpu_sc as plsc

F = 10000
V = 10000
E = 320000
D = 128
NC = 2
NS = 16
L = 16
DH = D // NC
EPT = E // NS
ECH = 80
NCH = EPT // ECH
UNR = 16
ROWS_PT = 624
TAIL = F - NS * ROWS_PT


def _sc_edge_aggregate(a_st, b_st, senders, receivers, edge_attr, w_row):
    mesh = plsc.VectorSubcoreMesh(core_axis_name="c", subcore_axis_name="s")
    cp = pltpu.CompilerParams()
    if "needs_layout_passes" in pltpu.CompilerParams.__dataclass_fields__:
        cp = dataclasses.replace(cp, needs_layout_passes=False)
    if "use_tc_tiling_on_sc" in pltpu.CompilerParams.__dataclass_fields__:
        cp = dataclasses.replace(cp, use_tc_tiling_on_sc=False)

    @functools.partial(
        pl.kernel,
        out_type=jax.ShapeDtypeStruct((NC, F, DH), jnp.float32),
        mesh=mesh,
        compiler_params=cp,
        scratch_types=[
            pltpu.VMEM((EPT,), jnp.int32),
            pltpu.VMEM((EPT,), jnp.int32),
            pltpu.VMEM((EPT,), jnp.float32),
            pltpu.VMEM((ECH,), jnp.int32),
            pltpu.VMEM((ECH,), jnp.int32),
            pltpu.VMEM((ECH,), jnp.int32),
            pltpu.VMEM((ECH, DH), jnp.float32),
            pltpu.VMEM((ECH, DH), jnp.float32),
            pltpu.VMEM((ECH, DH), jnp.float32),
            pltpu.VMEM((ECH, DH), jnp.float32),
            pltpu.VMEM((DH,), jnp.float32),
            pltpu.VMEM_SHARED((F, DH), jnp.float32),
            pltpu.SemaphoreType.DMA,
            pltpu.SemaphoreType.DMA,
        ],
    )
    def k(a_hbm, b_hbm, s_hbm, r_hbm, e_hbm, w_hbm, out_hbm,
          sidx, ridx, attr, radj0, radj1, ridx_c,
          abuf0, bbuf0, abuf1, bbuf1, wv, acc, sem0, sem1):
        cid = lax.axis_index("c")
        sid = lax.axis_index("s")

        pltpu.sync_copy(w_hbm.at[pl.ds(cid * DH, DH)], wv)
        ebase = sid * EPT
        pltpu.sync_copy(s_hbm.at[pl.ds(ebase, EPT)], sidx)
        pltpu.sync_copy(r_hbm.at[pl.ds(ebase, EPT)], ridx)
        pltpu.sync_copy(e_hbm.at[pl.ds(ebase, EPT)], attr)

        off = jnp.full((L,), cid * F, jnp.int32)

        @pl.loop(0, EPT // L)
        def _(i):
            sl = pl.ds(i * L, L)
            sidx[sl] = sidx[sl] + off

        zeros = jnp.zeros((L,), jnp.float32)

        @pl.loop(0, ECH * DH // L)
        def _(i):
            abuf0[i // (DH // L), pl.ds((i % (DH // L)) * L, L)] = zeros

        for j in range(ROWS_PT // ECH):
            pltpu.sync_copy(abuf0, acc.at[pl.ds(sid * ROWS_PT + j * ECH, ECH)])
        rrem = ROWS_PT % ECH
        if rrem:
            pltpu.sync_copy(
                abuf0.at[pl.ds(0, rrem)],
                acc.at[pl.ds(sid * ROWS_PT + ROWS_PT - rrem, rrem)])

        @pl.when(sid == 0)
        def _():
            pltpu.sync_copy(abuf0.at[pl.ds(0, TAIL)],
                            acc.at[pl.ds(NS * ROWS_PT, TAIL)])

        plsc.subcore_barrier()

        w_regs = [wv[pl.ds(g * L, L)] for g in range(DH // L)]

        def stage_issue(c, ab, bb, radj, sem):
            co = c * ECH
            for kk in range(ECH // L):
                sl = pl.ds(kk * L, L)
                radj[sl] = ridx[pl.ds(co + kk * L, L)] + off
            pltpu.async_copy(a_hbm.at[radj], ab, sem)
            pltpu.async_copy(b_hbm.at[sidx.at[pl.ds(co, ECH)]], bb, sem)

        def wait_g(c, ab, bb, radj, sem):
            co = c * ECH
            pltpu.make_async_copy(a_hbm.at[radj], ab, sem).wait()
            pltpu.make_async_copy(
                b_hbm.at[sidx.at[pl.ds(co, ECH)]], bb, sem).wait()

        def comp_scat(c, ab, bb):
            co = c * ECH

            @pl.loop(0, ECH // UNR)
            def _(u):
                eb = u * UNR
                for du in range(UNR):
                    e = eb + du
                    av = plsc.load_gather(
                        attr, [jnp.full((L,), co + e, jnp.int32)])
                    for g in range(DH // L):
                        sl = pl.ds(g * L, L)
                        m = ab[e, sl] + bb[e, sl] + av * w_regs[g]
                        ab[e, sl] = jnp.maximum(m, 0.0)

            for kk in range(ECH // L):
                sl = pl.ds(kk * L, L)
                ridx_c[sl] = ridx[pl.ds(co + kk * L, L)]
            pltpu.sync_copy(ab, acc.at[ridx_c], add=True)

        stage_issue(0, abuf0, bbuf0, radj0, sem0)
        stage_issue(1, abuf1, bbuf1, radj1, sem1)

        @pl.loop(0, NCH // 2 - 1)
        def _(t):
            c0 = 2 * t
            wait_g(c0, abuf0, bbuf0, radj0, sem0)
            comp_scat(c0, abuf0, bbuf0)
            stage_issue(c0 + 2, abuf0, bbuf0, radj0, sem0)
            wait_g(c0 + 1, abuf1, bbuf1, radj1, sem1)
            comp_scat(c0 + 1, abuf1, bbuf1)
            stage_issue(c0 + 3, abuf1, bbuf1, radj1, sem1)

        wait_g(NCH - 2, abuf0, bbuf0, radj0, sem0)
        comp_scat(NCH - 2, abuf0, bbuf0)
        wait_g(NCH - 1, abuf1, bbuf1, radj1, sem1)
        comp_scat(NCH - 1, abuf1, bbuf1)

        plsc.subcore_barrier()
        pltpu.sync_copy(acc.at[pl.ds(sid * ROWS_PT, ROWS_PT)],
                        out_hbm.at[cid, pl.ds(sid * ROWS_PT, ROWS_PT)])

        @pl.when(sid == 0)
        def _():
            pltpu.sync_copy(acc.at[pl.ds(NS * ROWS_PT, TAIL)],
                            out_hbm.at[cid, pl.ds(NS * ROWS_PT, TAIL)])

    return k(a_st, b_st, senders, receivers, edge_attr, w_row)


_BLK = 2000


def _tc_precompute(x_factors, x_variables, W1, W2, b_msg):
    def body(xf_ref, xv_ref, w1_ref, w2_ref, b_ref, a_ref, b_out_ref):
        a_ref[...] = jnp.dot(xf_ref[...], w1_ref[...],
                             preferred_element_type=jnp.float32,
                             precision=lax.Precision.HIGHEST) + b_ref[...]
        b_out_ref[...] = jnp.dot(xv_ref[...], w2_ref[...],
                                 preferred_element_type=jnp.float32,
                                 precision=lax.Precision.HIGHEST)

    return pl.pallas_call(
        body,
        grid=(F // _BLK,),
        in_specs=[
            pl.BlockSpec((_BLK, D), lambda i: (i, 0)),
            pl.BlockSpec((_BLK, D), lambda i: (i, 0)),
            pl.BlockSpec((D, D), lambda i: (0, 0)),
            pl.BlockSpec((D, D), lambda i: (0, 0)),
            pl.BlockSpec((1, D), lambda i: (0, 0)),
        ],
        out_specs=[
            pl.BlockSpec((_BLK, D), lambda i: (i, 0)),
            pl.BlockSpec((_BLK, D), lambda i: (i, 0)),
        ],
        out_shape=[
            jax.ShapeDtypeStruct((F, D), jnp.float32),
            jax.ShapeDtypeStruct((V, D), jnp.float32),
        ],
    )(x_factors, x_variables, W1, W2, b_msg.reshape(1, D))


def _tc_combine(x_factors, partials, Wc1, Wc2, b_comb):
    def body(xf_ref, p_ref, w1_ref, w2_ref, b_ref, o_ref):
        aggr = jnp.concatenate([p_ref[0], p_ref[1]], axis=-1)
        acc = jnp.dot(xf_ref[...], w1_ref[...],
                      preferred_element_type=jnp.float32,
                      precision=lax.Precision.HIGHEST)
        acc += jnp.dot(aggr, w2_ref[...],
                       preferred_element_type=jnp.float32,
                       precision=lax.Precision.HIGHEST)
        o_ref[...] = jnp.maximum(acc + b_ref[...], 0.0)

    return pl.pallas_call(
        body,
        grid=(F // _BLK,),
        in_specs=[
            pl.BlockSpec((_BLK, D), lambda i: (i, 0)),
            pl.BlockSpec((NC, _BLK, DH), lambda i: (0, i, 0)),
            pl.BlockSpec((D, D), lambda i: (0, 0)),
            pl.BlockSpec((D, D), lambda i: (0, 0)),
            pl.BlockSpec((1, D), lambda i: (0, 0)),
        ],
        out_specs=pl.BlockSpec((_BLK, D), lambda i: (i, 0)),
        out_shape=jax.ShapeDtypeStruct((F, D), jnp.float32),
    )(x_factors, partials, Wc1, Wc2, b_comb.reshape(1, D))


def kernel(x_variables, x_factors, senders, receivers, edge_attr,
           W_msg, b_msg, W_comb, b_comb):
    W1 = W_msg[:D]
    W2 = W_msg[D:2 * D]
    w_row = W_msg[2 * D]
    A, B = _tc_precompute(x_factors, x_variables, W1, W2, b_msg)
    a_st = jnp.concatenate([A[:, :DH], A[:, DH:]], axis=0)
    b_st = jnp.concatenate([B[:, :DH], B[:, DH:]], axis=0)
    partials = _sc_edge_aggregate(
        a_st, b_st,
        senders.astype(jnp.int32), receivers.astype(jnp.int32),
        edge_attr, w_row)
    return _tc_combine(x_factors, partials, W_comb[:D], W_comb[D:], b_comb)
